# baseline (speedup 1.0000x reference)
import functools

import jax
import jax.numpy as jnp
from jax.experimental import pallas as pl
from jax.experimental.pallas import tpu as pltpu
from jax.experimental.pallas import tpu_sc as plsc


_MESH = plsc.ScalarSubcoreMesh(axis_name="c", num_cores=1)


@functools.partial(
    pl.kernel,
    mesh=_MESH,
    out_type=jax.ShapeDtypeStruct((1,), jnp.float32),
)
def _scalar_gather(x_hbm, out_hbm):
    pltpu.sync_copy(x_hbm, out_hbm)


def kernel(x):
    return _scalar_gather(x.reshape(1))[0].reshape(())
